# baseline (speedup 1.0000x reference)
import functools

import jax
import jax.numpy as jnp
from jax import lax
from jax.experimental import pallas as pl
from jax.experimental.pallas import tpu as pltpu
from jax.experimental.pallas import tpu_sc as plsc

EMBED_DIM = 64
SCALE = 8.0
LOWEST = 1e-13
T = 2


@jax.jit
def _emb_lookup(tokens, table):
    R, S = tokens.shape
    D = EMBED_DIM
    info = plsc.get_sparse_core_info()
    NW = info.num_cores * info.num_subcores
    rows_per_w = R // NW
    n_chunks = rows_per_w // T
    assert n_chunks * T == rows_per_w and n_chunks % 2 == 0

    mesh = plsc.VectorSubcoreMesh(core_axis_name="c", subcore_axis_name="s")

    @functools.partial(
        pl.kernel,
        mesh=mesh,
        out_type=jax.ShapeDtypeStruct((R, S, 2 * D), jnp.float32),
        compiler_params=pltpu.CompilerParams(use_tc_tiling_on_sc=False),
        scratch_types=[
            pltpu.VMEM((2, T, S), jnp.int32),
            pltpu.VMEM((2, T, S, 2 * D), jnp.float32),
            pltpu.SemaphoreType.DMA((2,)),
        ],
    )
    def emb(tok_hbm, table_hbm, out_hbm, idx_v, rows_v, gsem):
        wid = lax.axis_index("s") * info.num_cores + lax.axis_index("c")
        base = wid * rows_per_w

        def stage(g, buf):
            row0 = base + g * T
            pltpu.sync_copy(tok_hbm.at[pl.ds(row0, T)], idx_v.at[buf])
            for t in range(T):
                pltpu.async_copy(
                    table_hbm.at[idx_v.at[buf, t, pl.ds(0, 128)]],
                    rows_v.at[buf, t, pl.ds(0, 128)],
                    gsem.at[buf],
                )
                pltpu.async_copy(
                    table_hbm.at[idx_v.at[buf, t, pl.ds(128, S - 128)]],
                    rows_v.at[buf, t, pl.ds(128, S - 128)],
                    gsem.at[buf],
                )

        def process(g, buf):
            pltpu.make_async_copy(
                out_hbm.at[pl.ds(0, T)],
                rows_v.at[buf],
                gsem.at[buf],
            ).wait()

            for t in range(T):

                def scale4(r4, _, t=t):
                    r = r4 * 4
                    for u in range(4):
                        for j in range(D // 16):
                            sl = pl.ds(j * 16, 16)
                            rows_v[buf, t, r + u, sl] = (
                                rows_v[buf, t, r + u, sl] * SCALE + LOWEST
                            )
                    return 0

                lax.fori_loop(0, S // 4, scale4, 0)
            pltpu.sync_copy(
                rows_v.at[buf, :, :, pl.ds(0, D)],
                out_hbm.at[pl.ds(base + g * T, T), :, pl.ds(0, D)],
            )

        stage(0, 0)

        def pair_body(i, _):
            g0 = i * 2
            stage(g0 + 1, 1)
            process(g0, 0)

            @pl.when(g0 + 2 < n_chunks)
            def _():
                stage(g0 + 2, 0)

            process(g0 + 1, 1)
            return 0

        lax.fori_loop(0, n_chunks // 2, pair_body, 0)

    return emb(tokens, table)[..., :D]


def kernel(token_tensor, table):
    table_padded = jnp.pad(table, ((0, 0), (0, EMBED_DIM)))
    return _emb_lookup(token_tensor, table_padded)


# speedup vs baseline: 1.0368x; 1.0368x over previous
import functools

import jax
import jax.numpy as jnp
from jax import lax
from jax.experimental import pallas as pl
from jax.experimental.pallas import tpu as pltpu
from jax.experimental.pallas import tpu_sc as plsc

EMBED_DIM = 64
SCALE = 8.0
LOWEST = 1e-13
T = 4


@jax.jit
def _emb_lookup(tokens, table):
    R, S = tokens.shape
    D = table.shape[1]
    info = plsc.get_sparse_core_info()
    NW = info.num_cores * info.num_subcores
    rows_per_w = R // NW
    n_chunks = rows_per_w // T
    assert n_chunks * T == rows_per_w and n_chunks % 2 == 0

    mesh = plsc.VectorSubcoreMesh(core_axis_name="c", subcore_axis_name="s")

    @functools.partial(
        pl.kernel,
        mesh=mesh,
        out_type=jax.ShapeDtypeStruct((R, S, 2 * D), jnp.float32),
        compiler_params=pltpu.CompilerParams(use_tc_tiling_on_sc=False),
        scratch_types=[
            pltpu.VMEM((2, T, S), jnp.int32),
            pltpu.VMEM((2, T, S, D), jnp.float32),
            pltpu.SemaphoreType.DMA((2,)),
        ],
    )
    def emb(tok_hbm, table_hbm, out_hbm, idx_v, rows_v, gsem):
        wid = lax.axis_index("s") * info.num_cores + lax.axis_index("c")
        base = wid * rows_per_w

        def stage(g, buf):
            row0 = base + g * T
            pltpu.sync_copy(tok_hbm.at[pl.ds(row0, T)], idx_v.at[buf])
            for t in range(T):
                pltpu.async_copy(
                    table_hbm.at[idx_v.at[buf, t, pl.ds(0, 128)]],
                    rows_v.at[buf, t, pl.ds(0, 128)],
                    gsem.at[buf],
                )
                pltpu.async_copy(
                    table_hbm.at[idx_v.at[buf, t, pl.ds(128, S - 128)]],
                    rows_v.at[buf, t, pl.ds(128, S - 128)],
                    gsem.at[buf],
                )

        def process(g, buf):
            pltpu.make_async_copy(
                out_hbm.at[pl.ds(0, T), :, pl.ds(0, D)],
                rows_v.at[buf],
                gsem.at[buf],
            ).wait()

            for t in range(T):

                def scale4(r4, _, t=t):
                    r = r4 * 4
                    for u in range(4):
                        for j in range(D // 16):
                            sl = pl.ds(j * 16, 16)
                            rows_v[buf, t, r + u, sl] = (
                                rows_v[buf, t, r + u, sl] * SCALE + LOWEST
                            )
                    return 0

                lax.fori_loop(0, S // 4, scale4, 0)
            pltpu.sync_copy(
                rows_v.at[buf],
                out_hbm.at[pl.ds(base + g * T, T), :, pl.ds(0, D)],
            )

        stage(0, 0)

        def pair_body(i, _):
            g0 = i * 2
            stage(g0 + 1, 1)
            process(g0, 0)

            @pl.when(g0 + 2 < n_chunks)
            def _():
                stage(g0 + 2, 0)

            process(g0 + 1, 1)
            return 0

        lax.fori_loop(0, n_chunks // 2, pair_body, 0)

    return emb(tokens, table)[..., :D]


def kernel(token_tensor, table):
    return _emb_lookup(token_tensor, table)
